# Initial kernel scaffold; baseline (speedup 1.0000x reference)
import jax
import jax.numpy as jnp
from jax.experimental import pallas as pl


def kernel(x, edge_index, W1, b1, W2, b2, Wout, bout):
    raise NotImplementedError("write your pallas kernel here")


# speedup vs baseline: 3.6709x; 3.6709x over previous
import functools

import jax
import jax.numpy as jnp
from jax import lax
from jax.experimental import pallas as pl
from jax.experimental.pallas import tpu as pltpu
from jax.experimental.pallas import tpu_sc as plsc

N = 10000
NPAD = 10240
E = 160000
NTILES = 16
ERANGE = NPAD // 4
EACC = ERANGE + 128

_MESH = dict(core_axis_name="c", subcore_axis_name="s", num_cores=2,
             num_subcores=16)

_F32 = jnp.float32


def _iota16():
  return lax.iota(jnp.int32, 16)


def _remap(dst_v, sdst_v, j, range_base, range_len, dump_base):
  d = dst_v[pl.ds(j * 16, 16)]
  local = d - range_base
  oob = (local < 0) | (local >= range_len)
  dump = _iota16() + (dump_base + (j % 8) * 16)
  sdst_v[pl.ds(j * 16, 16)] = jnp.where(oob, dump, local)


def _edge_body(nblk, table_hbm, src_hbm, dst_hbm, out_hbm,
               src_v, gidx_v, dst_v, sdst_v, rows_v, zbuf_v, obuf_v, acc_sh,
               gsem):
  c = lax.axis_index("c")
  s = lax.axis_index("s")
  zeros16 = jnp.zeros((16,), _F32)
  K = 400
  EPT = E // NTILES

  def zfill(i, _):
    for j in range(8):
      zbuf_v[i, pl.ds(j * 16, 16)] = zeros16
    return 0
  lax.fori_loop(0, 8, zfill, 0)

  def block(it, _):
    b = it // 2
    sub = it - 2 * b
    tbase = pl.multiple_of(b * NPAD, 8)
    range_base = pl.multiple_of((2 * c + sub) * ERANGE, 8)
    for q in range(21):
      pltpu.sync_copy(zbuf_v, acc_sh.at[pl.ds(s * 168 + q * 8, 8)])
    plsc.subcore_barrier()

    def chunk(g, _):
      e0 = s * EPT + g * K
      pltpu.sync_copy(src_hbm.at[pl.ds(e0, K)], src_v)
      pltpu.sync_copy(dst_hbm.at[pl.ds(e0, K)], dst_v)

      def remap(j, _):
        gidx_v[pl.ds(j * 16, 16)] = src_v[pl.ds(j * 16, 16)] + tbase
        _remap(dst_v, sdst_v, j, range_base, ERANGE, ERANGE)
        return 0
      lax.fori_loop(0, K // 16, remap, 0)

      pltpu.async_copy(table_hbm.at[gidx_v], rows_v, gsem).wait()
      pltpu.sync_copy(rows_v, acc_sh.at[sdst_v], add=True)
      return 0
    lax.fori_loop(0, EPT // K, chunk, 0)
    plsc.subcore_barrier()

    r0 = s * 160
    pltpu.sync_copy(acc_sh.at[pl.ds(r0, 160)], obuf_v)
    ofs = pl.multiple_of(tbase + range_base + r0, 8)
    pltpu.sync_copy(obuf_v, out_hbm.at[pl.ds(ofs, 160)])
    plsc.subcore_barrier()
    return 0
  lax.fori_loop(0, 2 * nblk, block, 0)


@functools.cache
def _make_edge_call(nblk):
  return pl.kernel(
      functools.partial(_edge_body, nblk),
      out_type=jax.ShapeDtypeStruct((nblk * NPAD, 128), _F32),
      mesh=plsc.VectorSubcoreMesh(**_MESH),
      scratch_types=[
          pltpu.VMEM((400,), jnp.int32),
          pltpu.VMEM((400,), jnp.int32),
          pltpu.VMEM((400,), jnp.int32),
          pltpu.VMEM((400,), jnp.int32),
          pltpu.VMEM((400, 128), _F32),
          pltpu.VMEM((8, 128), _F32),
          pltpu.VMEM((160, 128), _F32),
          pltpu.VMEM_SHARED((EACC, 128), _F32),
          pltpu.SemaphoreType.DMA,
      ],
  )


def _edge_call_2(table, src, dst):
  return _make_edge_call(2)(table, src, dst)


def _edge_call_4(table, src, dst):
  return _make_edge_call(4)(table, src, dst)


def _prep_body(x_ref, deg_ref, xs_ref, dinv_ref):
  k = pl.program_id(1)
  deg = deg_ref[:, 0:1] + 1.0
  dinv = lax.rsqrt(deg)
  xs_ref[...] = x_ref[...] * dinv

  @pl.when(k == 0)
  def _():
    dinv_ref[...] = jnp.broadcast_to(dinv, dinv_ref.shape)


def _prep_call(x, deg):
  BR = 1024
  nb = NPAD // BR
  return pl.pallas_call(
      _prep_body,
      grid=(nb, 2),
      in_specs=[
          pl.BlockSpec((BR, 128), lambda i, k: (i, k)),
          pl.BlockSpec((BR, 128), lambda i, k: (i, 0)),
      ],
      out_specs=[
          pl.BlockSpec((BR, 128), lambda i, k: (k * nb + i, 0)),
          pl.BlockSpec((BR, 128), lambda i, k: (i, 0)),
      ],
      out_shape=[
          jax.ShapeDtypeStruct((2 * NPAD, 128), _F32),
          jax.ShapeDtypeStruct((NPAD, 128), _F32),
      ],
      compiler_params=pltpu.CompilerParams(
          dimension_semantics=("parallel", "arbitrary")),
  )(x, deg)


def _l1_body(acc_ref, xs_ref, dinv_ref, w_ref, b_ref, out_ref):
  k = pl.program_id(2)
  t = dinv_ref[...] * (acc_ref[...] + xs_ref[...])
  part = jnp.dot(t, w_ref[...], preferred_element_type=_F32)

  @pl.when(k == 0)
  def _():
    out_ref[...] = part

  @pl.when(k == 1)
  def _():
    h = out_ref[...] + part + b_ref[...]
    out_ref[...] = dinv_ref[...] * jnp.maximum(h, 0.0)


def _l1_call(acc1, xs1, dinv2d, W1, b1):
  BR = 1024
  nb = NPAD // BR
  return pl.pallas_call(
      _l1_body,
      grid=(nb, 4, 2),
      in_specs=[
          pl.BlockSpec((BR, 128), lambda i, j, k: (k * nb + i, 0)),
          pl.BlockSpec((BR, 128), lambda i, j, k: (k * nb + i, 0)),
          pl.BlockSpec((BR, 128), lambda i, j, k: (i, 0)),
          pl.BlockSpec((128, 128), lambda i, j, k: (k, j)),
          pl.BlockSpec((1, 128), lambda i, j, k: (0, j)),
      ],
      out_specs=pl.BlockSpec((BR, 128), lambda i, j, k: (j * nb + i, 0)),
      out_shape=jax.ShapeDtypeStruct((4 * NPAD, 128), _F32),
      compiler_params=pltpu.CompilerParams(
          dimension_semantics=("parallel", "parallel", "arbitrary")),
  )(acc1, xs1, dinv2d, W1, b1.reshape(1, -1))


def _l2_body(acc_ref, xs_ref, dinv_ref, w2_ref, b2_ref, wo_ref, bo_ref,
             out_ref, h_acc):
  k = pl.program_id(1)
  t = dinv_ref[...] * (acc_ref[...] + xs_ref[...])
  part = jnp.dot(t, w2_ref[...], preferred_element_type=_F32)

  @pl.when(k == 0)
  def _():
    h_acc[...] = part

  @pl.when(k > 0)
  def _():
    h_acc[...] += part

  @pl.when(k == 3)
  def _():
    h2 = jnp.maximum(h_acc[...] + b2_ref[...], 0.0)
    logits = jnp.dot(h2, wo_ref[...], preferred_element_type=_F32)
    logits = logits + bo_ref[...]
    m = jnp.max(logits, axis=1, keepdims=True)
    z = logits - m
    lse = jnp.log(jnp.sum(jnp.exp(z), axis=1, keepdims=True))
    out_ref[...] = z - lse


def _l2_call(acc2, xs2, dinv2d, W2, b2, Wout, bout):
  BR = 1024
  nb = NPAD // BR
  return pl.pallas_call(
      _l2_body,
      grid=(nb, 4),
      in_specs=[
          pl.BlockSpec((BR, 128), lambda i, k: (k * nb + i, 0)),
          pl.BlockSpec((BR, 128), lambda i, k: (k * nb + i, 0)),
          pl.BlockSpec((BR, 128), lambda i, k: (i, 0)),
          pl.BlockSpec((128, 512), lambda i, k: (k, 0)),
          pl.BlockSpec((1, 512), lambda i, k: (0, 0)),
          pl.BlockSpec((512, 64), lambda i, k: (0, 0)),
          pl.BlockSpec((1, 64), lambda i, k: (0, 0)),
      ],
      out_specs=pl.BlockSpec((BR, 64), lambda i, k: (i, 0)),
      out_shape=jax.ShapeDtypeStruct((N, 64), _F32),
      scratch_shapes=[pltpu.VMEM((BR, 512), _F32)],
      compiler_params=pltpu.CompilerParams(
          dimension_semantics=("parallel", "arbitrary")),
  )(acc2, xs2, dinv2d, W2, b2.reshape(1, -1), Wout, bout.reshape(1, -1))


def kernel(x, edge_index, W1, b1, W2, b2, Wout, bout):
  src = edge_index[0].astype(jnp.int32)
  dst = edge_index[1].astype(jnp.int32)

  ones_t = jnp.ones((NPAD, 128), _F32)
  deg = _make_edge_call(1)(ones_t, src, dst)
  xs1, dinv2d = _prep_call(x, deg)
  acc1 = _edge_call_2(xs1, src, dst)
  xs2 = _l1_call(acc1, xs1, dinv2d, W1, b1)
  acc2 = _edge_call_4(xs2, src, dst)
  return _l2_call(acc2, xs2, dinv2d, W2, b2, Wout, bout)


# speedup vs baseline: 3.9526x; 1.0767x over previous
import functools

import jax
import jax.numpy as jnp
from jax import lax
from jax.experimental import pallas as pl
from jax.experimental.pallas import tpu as pltpu
from jax.experimental.pallas import tpu_sc as plsc

N = 10000
NPAD = 10240
E = 160000
NTILES = 16
ERANGE = NPAD // 4
EACC = ERANGE + 128

_MESH = dict(core_axis_name="c", subcore_axis_name="s", num_cores=2,
             num_subcores=16)

_F32 = jnp.float32


def _iota16():
  return lax.iota(jnp.int32, 16)


def _remap(dst_v, sdst_v, j, range_base, range_len, dump_base):
  d = dst_v[pl.ds(j * 16, 16)]
  local = d - range_base
  oob = (local < 0) | (local >= range_len)
  dump = _iota16() + (dump_base + (j % 8) * 16)
  sdst_v[pl.ds(j * 16, 16)] = jnp.where(oob, dump, local)


def _edge_body(nblk, do_gather, table_hbm, src_hbm, dst_hbm, out_hbm,
               src_v, gidx_v, dst_v, sdst_v, rows_v, zbuf_v, obuf_v, acc_sh,
               gsem):
  c = lax.axis_index("c")
  s = lax.axis_index("s")
  zeros16 = jnp.zeros((16,), _F32)
  ones16 = jnp.ones((16,), _F32)
  K = 400
  EPT = E // NTILES

  def zfill(i, _):
    for j in range(8):
      zbuf_v[i, pl.ds(j * 16, 16)] = zeros16
    return 0
  lax.fori_loop(0, 8, zfill, 0)

  if not do_gather:
    def ofill(i, _):
      for j in range(8):
        rows_v[i, pl.ds(j * 16, 16)] = ones16
      return 0
    lax.fori_loop(0, K, ofill, 0)

  def block(it, _):
    b = it // 2
    sub = it - 2 * b
    tbase = pl.multiple_of(b * NPAD, 8)
    range_base = pl.multiple_of((2 * c + sub) * ERANGE, 8)
    for q in range(21):
      pltpu.sync_copy(zbuf_v, acc_sh.at[pl.ds(s * 168 + q * 8, 8)])
    plsc.subcore_barrier()

    def chunk(g, _):
      e0 = s * EPT + g * K
      if do_gather:
        pltpu.sync_copy(src_hbm.at[pl.ds(e0, K)], src_v)
      pltpu.sync_copy(dst_hbm.at[pl.ds(e0, K)], dst_v)

      def remap(j, _):
        if do_gather:
          gidx_v[pl.ds(j * 16, 16)] = src_v[pl.ds(j * 16, 16)] + tbase
        _remap(dst_v, sdst_v, j, range_base, ERANGE, ERANGE)
        return 0
      lax.fori_loop(0, K // 16, remap, 0)

      if do_gather:
        pltpu.async_copy(table_hbm.at[gidx_v], rows_v, gsem).wait()
      pltpu.sync_copy(rows_v, acc_sh.at[sdst_v], add=True)
      return 0
    lax.fori_loop(0, EPT // K, chunk, 0)
    plsc.subcore_barrier()

    r0 = s * 160
    pltpu.sync_copy(acc_sh.at[pl.ds(r0, 160)], obuf_v)
    ofs = pl.multiple_of(tbase + range_base + r0, 8)
    pltpu.sync_copy(obuf_v, out_hbm.at[pl.ds(ofs, 160)])
    plsc.subcore_barrier()
    return 0
  lax.fori_loop(0, 2 * nblk, block, 0)


@functools.cache
def _make_edge_call(nblk, do_gather=True):
  return pl.kernel(
      functools.partial(_edge_body, nblk, do_gather),
      out_type=jax.ShapeDtypeStruct((nblk * NPAD, 128), _F32),
      mesh=plsc.VectorSubcoreMesh(**_MESH),
      scratch_types=[
          pltpu.VMEM((400,), jnp.int32),
          pltpu.VMEM((400,), jnp.int32),
          pltpu.VMEM((400,), jnp.int32),
          pltpu.VMEM((400,), jnp.int32),
          pltpu.VMEM((400, 128), _F32),
          pltpu.VMEM((8, 128), _F32),
          pltpu.VMEM((160, 128), _F32),
          pltpu.VMEM_SHARED((EACC, 128), _F32),
          pltpu.SemaphoreType.DMA,
      ],
  )


def _edge_call_2(table, src, dst):
  return _make_edge_call(2)(table, src, dst)


def _edge_call_4(table, src, dst):
  return _make_edge_call(4)(table, src, dst)


def _prep_body(x_ref, deg_ref, xs_ref, dinv_ref):
  k = pl.program_id(1)
  deg = deg_ref[:, 0:1] + 1.0
  dinv = lax.rsqrt(deg)
  xs_ref[...] = x_ref[...] * dinv

  @pl.when(k == 0)
  def _():
    dinv_ref[...] = jnp.broadcast_to(dinv, dinv_ref.shape)


def _prep_call(x, deg):
  BR = 1024
  nb = NPAD // BR
  return pl.pallas_call(
      _prep_body,
      grid=(nb, 2),
      in_specs=[
          pl.BlockSpec((BR, 128), lambda i, k: (i, k)),
          pl.BlockSpec((BR, 128), lambda i, k: (i, 0)),
      ],
      out_specs=[
          pl.BlockSpec((BR, 128), lambda i, k: (k * nb + i, 0)),
          pl.BlockSpec((BR, 128), lambda i, k: (i, 0)),
      ],
      out_shape=[
          jax.ShapeDtypeStruct((2 * NPAD, 128), _F32),
          jax.ShapeDtypeStruct((NPAD, 128), _F32),
      ],
      compiler_params=pltpu.CompilerParams(
          dimension_semantics=("parallel", "arbitrary")),
  )(x, deg)


def _l1_body(acc_ref, xs_ref, dinv_ref, w_ref, b_ref, out_ref):
  k = pl.program_id(2)
  t = dinv_ref[...] * (acc_ref[...] + xs_ref[...])
  part = jnp.dot(t, w_ref[...], preferred_element_type=_F32)

  @pl.when(k == 0)
  def _():
    out_ref[...] = part

  @pl.when(k == 1)
  def _():
    h = out_ref[...] + part + b_ref[...]
    out_ref[...] = dinv_ref[...] * jnp.maximum(h, 0.0)


def _l1_call(acc1, xs1, dinv2d, W1, b1):
  BR = 1024
  nb = NPAD // BR
  return pl.pallas_call(
      _l1_body,
      grid=(nb, 4, 2),
      in_specs=[
          pl.BlockSpec((BR, 128), lambda i, j, k: (k * nb + i, 0)),
          pl.BlockSpec((BR, 128), lambda i, j, k: (k * nb + i, 0)),
          pl.BlockSpec((BR, 128), lambda i, j, k: (i, 0)),
          pl.BlockSpec((128, 128), lambda i, j, k: (k, j)),
          pl.BlockSpec((1, 128), lambda i, j, k: (0, j)),
      ],
      out_specs=pl.BlockSpec((BR, 128), lambda i, j, k: (j * nb + i, 0)),
      out_shape=jax.ShapeDtypeStruct((4 * NPAD, 128), _F32),
      compiler_params=pltpu.CompilerParams(
          dimension_semantics=("parallel", "parallel", "arbitrary")),
  )(acc1, xs1, dinv2d, W1, b1.reshape(1, -1))


def _l2_body(acc_ref, xs_ref, dinv_ref, w2_ref, b2_ref, wo_ref, bo_ref,
             out_ref, h_acc):
  k = pl.program_id(1)
  t = dinv_ref[...] * (acc_ref[...] + xs_ref[...])
  part = jnp.dot(t, w2_ref[...], preferred_element_type=_F32)

  @pl.when(k == 0)
  def _():
    h_acc[...] = part

  @pl.when(k > 0)
  def _():
    h_acc[...] += part

  @pl.when(k == 3)
  def _():
    h2 = jnp.maximum(h_acc[...] + b2_ref[...], 0.0)
    logits = jnp.dot(h2, wo_ref[...], preferred_element_type=_F32)
    logits = logits + bo_ref[...]
    m = jnp.max(logits, axis=1, keepdims=True)
    z = logits - m
    lse = jnp.log(jnp.sum(jnp.exp(z), axis=1, keepdims=True))
    out_ref[...] = z - lse


def _l2_call(acc2, xs2, dinv2d, W2, b2, Wout, bout):
  BR = 1024
  nb = NPAD // BR
  return pl.pallas_call(
      _l2_body,
      grid=(nb, 4),
      in_specs=[
          pl.BlockSpec((BR, 128), lambda i, k: (k * nb + i, 0)),
          pl.BlockSpec((BR, 128), lambda i, k: (k * nb + i, 0)),
          pl.BlockSpec((BR, 128), lambda i, k: (i, 0)),
          pl.BlockSpec((128, 512), lambda i, k: (k, 0)),
          pl.BlockSpec((1, 512), lambda i, k: (0, 0)),
          pl.BlockSpec((512, 64), lambda i, k: (0, 0)),
          pl.BlockSpec((1, 64), lambda i, k: (0, 0)),
      ],
      out_specs=pl.BlockSpec((BR, 64), lambda i, k: (i, 0)),
      out_shape=jax.ShapeDtypeStruct((N, 64), _F32),
      scratch_shapes=[pltpu.VMEM((BR, 512), _F32)],
      compiler_params=pltpu.CompilerParams(
          dimension_semantics=("parallel", "arbitrary")),
  )(acc2, xs2, dinv2d, W2, b2.reshape(1, -1), Wout, bout.reshape(1, -1))


def kernel(x, edge_index, W1, b1, W2, b2, Wout, bout):
  src = edge_index[0].astype(jnp.int32)
  dst = edge_index[1].astype(jnp.int32)

  ones_t = jnp.ones((8, 128), _F32)
  deg = _make_edge_call(1, False)(ones_t, src, dst)
  xs1, dinv2d = _prep_call(x, deg)
  acc1 = _edge_call_2(xs1, src, dst)
  xs2 = _l1_call(acc1, xs1, dinv2d, W1, b1)
  acc2 = _edge_call_4(xs2, src, dst)
  return _l2_call(acc2, xs2, dinv2d, W2, b2, Wout, bout)
